# baseline (device time: 15199 ns/iter reference)
import jax
import jax.numpy as jnp
from jax import lax
from jax.experimental import pallas as pl
from jax.experimental.pallas import tpu as pltpu

N_DEV = 32
KW = 4
HALO = KW - 1


def kernel(x, k):
    b, s, c = x.shape

    def body(x_any, k_ref, out_any,
             x_vmem, out_vmem, halo_ref,
             in_sem, out_sem, send_sem, recv_sem, ack_sem):
        my = lax.axis_index("i")
        has_left = my > 0
        has_right = my < N_DEV - 1

        in_dma = pltpu.make_async_copy(x_any, x_vmem, in_sem)
        in_dma.start()

        barrier_sem = pltpu.get_barrier_semaphore()

        @pl.when(has_left)
        def _():
            pl.semaphore_signal(barrier_sem, inc=1, device_id=(my - 1,),
                                device_id_type=pl.DeviceIdType.MESH)

        @pl.when(has_right)
        def _():
            pl.semaphore_signal(barrier_sem, inc=1, device_id=(my + 1,),
                                device_id_type=pl.DeviceIdType.MESH)

        n_nbrs = has_left.astype(jnp.int32) + has_right.astype(jnp.int32)
        pl.semaphore_wait(barrier_sem, n_nbrs)

        rdma = pltpu.make_async_remote_copy(
            src_ref=x_any.at[:, pl.ds(s - HALO, HALO), :],
            dst_ref=halo_ref,
            send_sem=send_sem,
            recv_sem=recv_sem,
            device_id=((my + 1) % N_DEV,),
            device_id_type=pl.DeviceIdType.MESH,
        )

        @pl.when(has_right)
        def _():
            rdma.start()

        in_dma.wait()

        xv = x_vmem[...].astype(jnp.bfloat16)
        kv = k_ref[...].astype(jnp.bfloat16)
        pad = jnp.concatenate([jnp.zeros((b, HALO, c), jnp.bfloat16), xv], axis=1)
        out = jnp.zeros((b, s, c), jnp.bfloat16)
        for t in range(KW):
            out = out + pad[:, t:t + s, :] * kv[t][None, None, :]
        out_vmem[...] = out * jax.nn.sigmoid(out)

        @pl.when(has_left)
        def _():
            rdma.wait_recv()
            pl.semaphore_signal(ack_sem, inc=1, device_id=(my - 1,),
                                device_id_type=pl.DeviceIdType.MESH)
            halo = halo_ref[...].astype(jnp.bfloat16)
            small = jnp.concatenate([halo, xv[:, :HALO, :]], axis=1)
            fix = jnp.zeros((b, HALO, c), jnp.bfloat16)
            for t in range(KW):
                fix = fix + small[:, t:t + HALO, :] * kv[t][None, None, :]
            out_vmem[:, :HALO, :] = fix * jax.nn.sigmoid(fix)

        out_dma = pltpu.make_async_copy(out_vmem, out_any, out_sem)
        out_dma.start()

        @pl.when(has_right)
        def _():
            rdma.wait_send()
            pl.semaphore_wait(ack_sem, 1)

        out_dma.wait()

    return pl.pallas_call(
        body,
        out_shape=jax.ShapeDtypeStruct((b, s, c), jnp.bfloat16),
        in_specs=[
            pl.BlockSpec(memory_space=pl.ANY),
            pl.BlockSpec(memory_space=pltpu.VMEM),
        ],
        out_specs=pl.BlockSpec(memory_space=pl.ANY),
        scratch_shapes=[
            pltpu.VMEM((b, s, c), x.dtype),
            pltpu.VMEM((b, s, c), jnp.bfloat16),
            pltpu.VMEM((b, HALO, c), x.dtype),
            pltpu.SemaphoreType.DMA,
            pltpu.SemaphoreType.DMA,
            pltpu.SemaphoreType.DMA,
            pltpu.SemaphoreType.DMA,
            pltpu.SemaphoreType.REGULAR,
        ],
        compiler_params=pltpu.CompilerParams(collective_id=0),
    )(x, k)


# device time: 14593 ns/iter; 1.0415x vs baseline; 1.0415x over previous
import jax
import jax.numpy as jnp
from jax import lax
from jax.experimental import pallas as pl
from jax.experimental.pallas import tpu as pltpu

N_DEV = 32
KW = 4
HALO = KW - 1
NC = 4


def kernel(x, k):
    b, s, c = x.shape
    cs = s // NC

    def body(x_any, k_any, out_any,
             x_vmem, out_vmem, k_vmem, halo_ref,
             in_sems, out_sems, k_sem, send_sem, recv_sem, ack_sem):
        my = lax.axis_index("i")
        has_left = my > 0
        has_right = my < N_DEV - 1

        k_dma = pltpu.make_async_copy(k_any, k_vmem, k_sem)
        k_dma.start()
        in_dmas = []
        for j in range(NC):
            d = pltpu.make_async_copy(
                x_any.at[:, pl.ds(j * cs, cs), :],
                x_vmem.at[:, pl.ds(j * cs, cs), :],
                in_sems.at[j],
            )
            d.start()
            in_dmas.append(d)

        barrier_sem = pltpu.get_barrier_semaphore()

        @pl.when(has_left)
        def _():
            pl.semaphore_signal(barrier_sem, inc=1, device_id=(my - 1,),
                                device_id_type=pl.DeviceIdType.MESH)

        @pl.when(has_right)
        def _():
            pl.semaphore_signal(barrier_sem, inc=1, device_id=(my + 1,),
                                device_id_type=pl.DeviceIdType.MESH)

        n_nbrs = has_left.astype(jnp.int32) + has_right.astype(jnp.int32)
        pl.semaphore_wait(barrier_sem, n_nbrs)

        rdma = pltpu.make_async_remote_copy(
            src_ref=x_any.at[:, pl.ds(s - HALO, HALO), :],
            dst_ref=halo_ref,
            send_sem=send_sem,
            recv_sem=recv_sem,
            device_id=((my + 1) % N_DEV,),
            device_id_type=pl.DeviceIdType.MESH,
        )

        @pl.when(has_right)
        def _():
            rdma.start()

        k_dma.wait()
        kv_holder = []

        def compute_chunk(j, left):
            kv = kv_holder[0]
            xj = x_vmem[:, j * cs:(j + 1) * cs, :].astype(jnp.bfloat16)
            pad = jnp.concatenate([left, xj], axis=1)
            o = jnp.zeros((b, cs, c), jnp.bfloat16)
            for t in range(KW):
                o = o + pad[:, t:t + cs, :] * kv[t][None, None, :]
            out_vmem[:, j * cs:(j + 1) * cs, :] = o * jax.nn.sigmoid(o)

        out_dmas = [
            pltpu.make_async_copy(
                out_vmem.at[:, pl.ds(j * cs, cs), :],
                out_any.at[:, pl.ds(j * cs, cs), :],
                out_sems.at[j],
            )
            for j in range(NC)
        ]

        in_dmas[0].wait()
        for j in range(1, NC):
            in_dmas[j].wait()
            if j == 1:
                kv_holder.append(k_vmem[...].astype(jnp.bfloat16))
            left = x_vmem[:, j * cs - HALO:j * cs, :].astype(jnp.bfloat16)
            compute_chunk(j, left)
            out_dmas[j].start()

        @pl.when(has_left)
        def _():
            rdma.wait_recv()
            pl.semaphore_signal(ack_sem, inc=1, device_id=(my - 1,),
                                device_id_type=pl.DeviceIdType.MESH)
            compute_chunk(0, halo_ref[...].astype(jnp.bfloat16))
            out_dmas[0].start()

        @pl.when(jnp.logical_not(has_left))
        def _():
            compute_chunk(0, jnp.zeros((b, HALO, c), jnp.bfloat16))
            out_dmas[0].start()

        @pl.when(has_right)
        def _():
            rdma.wait_send()
            pl.semaphore_wait(ack_sem, 1)

        for d in out_dmas:
            d.wait()

    hbm = pl.BlockSpec(memory_space=pltpu.MemorySpace.HBM)
    return pl.pallas_call(
        body,
        out_shape=jax.ShapeDtypeStruct((b, s, c), jnp.bfloat16),
        in_specs=[hbm, hbm],
        out_specs=hbm,
        scratch_shapes=[
            pltpu.VMEM((b, s, c), x.dtype),
            pltpu.VMEM((b, s, c), jnp.bfloat16),
            pltpu.VMEM(k.shape, k.dtype),
            pltpu.VMEM((b, HALO, c), x.dtype),
            pltpu.SemaphoreType.DMA((NC,)),
            pltpu.SemaphoreType.DMA((NC,)),
            pltpu.SemaphoreType.DMA,
            pltpu.SemaphoreType.DMA,
            pltpu.SemaphoreType.DMA,
            pltpu.SemaphoreType.REGULAR,
        ],
        compiler_params=pltpu.CompilerParams(collective_id=0),
    )(x, k)


# device time: 13459 ns/iter; 1.1293x vs baseline; 1.0843x over previous
import jax
import jax.numpy as jnp
from jax import lax
from jax.experimental import pallas as pl
from jax.experimental.pallas import tpu as pltpu

N_DEV = 32
KW = 4
HALO = KW - 1


def kernel(x, k):
    b, s, c = x.shape

    def body(x_ref, k_ref, out_ref, halo_ref, send_sem, recv_sem):
        my = lax.axis_index("i")
        has_left = my > 0
        has_right = my < N_DEV - 1

        barrier_sem = pltpu.get_barrier_semaphore()

        @pl.when(has_left)
        def _():
            pl.semaphore_signal(barrier_sem, inc=1, device_id=(my - 1,),
                                device_id_type=pl.DeviceIdType.MESH)

        rdma = pltpu.make_async_remote_copy(
            src_ref=x_ref.at[:, pl.ds(s - HALO, HALO), :],
            dst_ref=halo_ref,
            send_sem=send_sem,
            recv_sem=recv_sem,
            device_id=((my + 1) % N_DEV,),
            device_id_type=pl.DeviceIdType.MESH,
        )

        @pl.when(has_right)
        def _():
            pl.semaphore_wait(barrier_sem, 1)
            rdma.start()

        xv = x_ref[...].astype(jnp.bfloat16)
        kv = k_ref[...].astype(jnp.bfloat16)
        pad = jnp.concatenate([jnp.zeros((b, HALO, c), jnp.bfloat16), xv], axis=1)
        out = jnp.zeros((b, s, c), jnp.bfloat16)
        for t in range(KW):
            out = out + pad[:, t:t + s, :] * kv[t][None, None, :]
        out_ref[...] = out * jax.nn.sigmoid(out)

        @pl.when(has_right)
        def _():
            rdma.wait_send()

        @pl.when(has_left)
        def _():
            rdma.wait_recv()
            halo = halo_ref[...].astype(jnp.bfloat16)
            small = jnp.concatenate([halo, xv[:, :HALO, :]], axis=1)
            fix = jnp.zeros((b, HALO, c), jnp.bfloat16)
            for t in range(KW):
                fix = fix + small[:, t:t + HALO, :] * kv[t][None, None, :]
            out_ref[:, :HALO, :] = fix * jax.nn.sigmoid(fix)

    return pl.pallas_call(
        body,
        out_shape=jax.ShapeDtypeStruct((b, s, c), jnp.bfloat16),
        in_specs=[
            pl.BlockSpec(memory_space=pltpu.VMEM),
            pl.BlockSpec(memory_space=pltpu.VMEM),
        ],
        out_specs=pl.BlockSpec(memory_space=pltpu.VMEM),
        scratch_shapes=[
            pltpu.VMEM((b, HALO, c), x.dtype),
            pltpu.SemaphoreType.DMA,
            pltpu.SemaphoreType.DMA,
        ],
        compiler_params=pltpu.CompilerParams(collective_id=0),
    )(x, k)


# device time: 12232 ns/iter; 1.2426x vs baseline; 1.1003x over previous
import jax
import jax.numpy as jnp
from jax import lax
from jax.experimental import pallas as pl
from jax.experimental.pallas import tpu as pltpu

N_DEV = 32
KW = 4
HALO = KW - 1


def kernel(x, k):
    b, s, c = x.shape

    def body(x_ref, k_ref, out_ref, halo_ref, send_sem, recv_sem):
        my = lax.axis_index("i")
        has_left = my > 0
        has_right = my < N_DEV - 1

        barrier_sem = pltpu.get_barrier_semaphore()

        @pl.when(has_left)
        def _():
            pl.semaphore_signal(barrier_sem, inc=1, device_id=(my - 1,),
                                device_id_type=pl.DeviceIdType.MESH)

        rdma = pltpu.make_async_remote_copy(
            src_ref=x_ref.at[:, pl.ds(s - HALO, HALO), :],
            dst_ref=halo_ref,
            send_sem=send_sem,
            recv_sem=recv_sem,
            device_id=((my + 1) % N_DEV,),
            device_id_type=pl.DeviceIdType.MESH,
        )

        xv = x_ref[...].astype(jnp.bfloat16)
        kv = k_ref[...].astype(jnp.bfloat16)
        pad = jnp.concatenate([jnp.zeros((b, HALO, c), jnp.bfloat16), xv], axis=1)

        out = pad[:, 0:s, :] * kv[0][None, None, :]

        @pl.when(has_right)
        def _():
            pl.semaphore_wait(barrier_sem, 1)
            rdma.start()

        for t in range(1, KW):
            out = out + pad[:, t:t + s, :] * kv[t][None, None, :]
        out_ref[...] = out * jax.nn.sigmoid(out)

        @pl.when(has_right)
        def _():
            rdma.wait_send()

        @pl.when(has_left)
        def _():
            rdma.wait_recv()
            halo = halo_ref[...].astype(jnp.bfloat16)
            small = jnp.concatenate([halo, xv[:, :HALO, :]], axis=1)
            fix = jnp.zeros((b, HALO, c), jnp.bfloat16)
            for t in range(KW):
                fix = fix + small[:, t:t + HALO, :] * kv[t][None, None, :]
            out_ref[:, :HALO, :] = fix * jax.nn.sigmoid(fix)

    return pl.pallas_call(
        body,
        out_shape=jax.ShapeDtypeStruct((b, s, c), jnp.bfloat16),
        in_specs=[
            pl.BlockSpec(memory_space=pltpu.VMEM),
            pl.BlockSpec(memory_space=pltpu.VMEM),
        ],
        out_specs=pl.BlockSpec(memory_space=pltpu.VMEM),
        scratch_shapes=[
            pltpu.VMEM((b, HALO, c), x.dtype),
            pltpu.SemaphoreType.DMA,
            pltpu.SemaphoreType.DMA,
        ],
        compiler_params=pltpu.CompilerParams(collective_id=0),
    )(x, k)
